# baseline (speedup 1.0000x reference)
import jax
import jax.numpy as jnp
from jax import lax
from jax.experimental import pallas as pl
from jax.experimental.pallas import tpu as pltpu

B, T, V, S = 32, 64, 10000, 512
COV_LOSS_WT = 1.0
BT = B * T

_CB = 16
_ROWS = _CB * T


def _fused_body(tgt_s, fd_any, tgtv_ref, mask3_ref, attn_ref,
                out_ref, buf_ref, acc_ref, sem):
    c = pl.program_id(0)

    @pl.when(c == 0)
    def _():
        acc_ref[...] = jnp.zeros((1, 1), jnp.float32)

    row = lax.broadcasted_iota(jnp.int32, (T, T), 0)
    col = lax.broadcasted_iota(jnp.int32, (T, T), 1)
    ltri = (col < row).astype(jnp.float32)
    for bb in range(_CB):
        for m in range(T):
            jj = bb * T + m
            p = c * _ROWS + jj
            tt = tgt_s[p]
            c128 = pl.multiple_of((tt // 128) * 128, 128)
            pltpu.make_async_copy(
                fd_any.at[p, pl.ds(c128, 128)], buf_ref.at[jj], sem).start()
        attn = attn_ref[bb]
        coverage = jnp.dot(ltri, attn,
                           preferred_element_type=jnp.float32)
        covloss = jnp.sum(jnp.minimum(attn, coverage), axis=1,
                          keepdims=True)
        mrow = mask3_ref[bb]
        s_cov = jnp.dot(mrow, covloss,
                        preferred_element_type=jnp.float32)
        dl = jnp.sum(mrow, axis=1, keepdims=True)
        acc_ref[...] += COV_LOSS_WT * s_cov / dl

    pltpu.make_async_copy(
        fd_any.at[pl.ds(0, _ROWS), pl.ds(0, 128)], buf_ref, sem).wait()

    lanes = lax.broadcasted_iota(jnp.int32, (_ROWS, 128), 1)
    sel = lanes == (tgtv_ref[...] & 127)
    picked = jnp.where(sel, buf_ref[...],
                       jnp.zeros((_ROWS, 128), jnp.float32))
    gold = jnp.sum(picked, axis=1, keepdims=True)
    nlog = -jnp.log(gold)
    for bb in range(_CB):
        ncol = nlog[bb * T:(bb + 1) * T]
        mrow = mask3_ref[bb]
        s_nll = jnp.dot(mrow, ncol,
                        preferred_element_type=jnp.float32)
        dl = jnp.sum(mrow, axis=1, keepdims=True)
        acc_ref[...] += s_nll / dl

    @pl.when(c == B // _CB - 1)
    def _():
        out_ref[...] = acc_ref[...] / B


def _fused(tgt_flat, fd2, tgtv, mask3, attn, interpret=False):
    return pl.pallas_call(
        _fused_body,
        grid=(B // _CB,),
        in_specs=[
            pl.BlockSpec(memory_space=pltpu.SMEM),
            pl.BlockSpec(memory_space=pltpu.MemorySpace.HBM),
            pl.BlockSpec((_ROWS, 1), lambda c: (c, 0)),
            pl.BlockSpec((_CB, 1, T), lambda c: (c, 0, 0)),
            pl.BlockSpec((_CB, T, S), lambda c: (c, 0, 0)),
        ],
        out_specs=pl.BlockSpec((1, 1), lambda c: (0, 0)),
        out_shape=jax.ShapeDtypeStruct((1, 1), jnp.float32),
        scratch_shapes=[
            pltpu.VMEM((_ROWS, 128), jnp.float32),
            pltpu.VMEM((1, 1), jnp.float32),
            pltpu.SemaphoreType.DMA,
        ],
        interpret=interpret,
    )(tgt_flat, fd2, tgtv, mask3, attn)


def kernel(final_dists, attn_dists, target_batch, dec_padding_mask):
    tgt_flat = target_batch.reshape(-1)
    out = _fused(tgt_flat,
                 final_dists.reshape(BT, V),
                 target_batch.reshape(BT, 1),
                 dec_padding_mask.reshape(B, 1, T),
                 attn_dists)
    return out.reshape(())


# speedup vs baseline: 1.1072x; 1.1072x over previous
import jax
import jax.numpy as jnp
from jax import lax
from jax.experimental import pallas as pl
from jax.experimental.pallas import tpu as pltpu

B, T, V, S = 32, 64, 10000, 512
COV_LOSS_WT = 1.0
BT = B * T

_CB = 32
_ROWS = _CB * T


def _fused_body(tgt_s, fd_any, tgtv_ref, mask3_ref, attn_ref,
                out_ref, buf_ref, acc_ref, sem):
    c = pl.program_id(0)

    @pl.when(c == 0)
    def _():
        acc_ref[...] = jnp.zeros((1, 1), jnp.float32)

    row = lax.broadcasted_iota(jnp.int32, (T, T), 0)
    col = lax.broadcasted_iota(jnp.int32, (T, T), 1)
    ltri = (col < row).astype(jnp.float32)
    for bb in range(_CB):
        for m in range(T):
            jj = bb * T + m
            p = c * _ROWS + jj
            tt = tgt_s[p]
            c128 = pl.multiple_of((tt // 128) * 128, 128)
            pltpu.make_async_copy(
                fd_any.at[p, pl.ds(c128, 128)], buf_ref.at[jj], sem).start()
        attn = attn_ref[bb]
        coverage = jnp.dot(ltri, attn,
                           preferred_element_type=jnp.float32)
        covloss = jnp.sum(jnp.minimum(attn, coverage), axis=1,
                          keepdims=True)
        mrow = mask3_ref[bb]
        s_cov = jnp.dot(mrow, covloss,
                        preferred_element_type=jnp.float32)
        dl = jnp.sum(mrow, axis=1, keepdims=True)
        acc_ref[...] += COV_LOSS_WT * s_cov / dl

    pltpu.make_async_copy(
        fd_any.at[pl.ds(0, _ROWS), pl.ds(0, 128)], buf_ref, sem).wait()

    lanes = lax.broadcasted_iota(jnp.int32, (_ROWS, 128), 1)
    sel = lanes == (tgtv_ref[...] & 127)
    picked = jnp.where(sel, buf_ref[...],
                       jnp.zeros((_ROWS, 128), jnp.float32))
    gold = jnp.sum(picked, axis=1, keepdims=True)
    nlog = -jnp.log(gold)
    for bb in range(_CB):
        ncol = nlog[bb * T:(bb + 1) * T]
        mrow = mask3_ref[bb]
        s_nll = jnp.dot(mrow, ncol,
                        preferred_element_type=jnp.float32)
        dl = jnp.sum(mrow, axis=1, keepdims=True)
        acc_ref[...] += s_nll / dl

    @pl.when(c == B // _CB - 1)
    def _():
        out_ref[...] = acc_ref[...] / B


def _fused(tgt_flat, fd2, tgtv, mask3, attn, interpret=False):
    return pl.pallas_call(
        _fused_body,
        grid=(B // _CB,),
        in_specs=[
            pl.BlockSpec(memory_space=pltpu.SMEM),
            pl.BlockSpec(memory_space=pltpu.MemorySpace.HBM),
            pl.BlockSpec((_ROWS, 1), lambda c: (c, 0)),
            pl.BlockSpec((_CB, 1, T), lambda c: (c, 0, 0)),
            pl.BlockSpec((_CB, T, S), lambda c: (c, 0, 0)),
        ],
        out_specs=pl.BlockSpec((1, 1), lambda c: (0, 0)),
        out_shape=jax.ShapeDtypeStruct((1, 1), jnp.float32),
        scratch_shapes=[
            pltpu.VMEM((_ROWS, 128), jnp.float32),
            pltpu.VMEM((1, 1), jnp.float32),
            pltpu.SemaphoreType.DMA,
        ],
        interpret=interpret,
    )(tgt_flat, fd2, tgtv, mask3, attn)


def kernel(final_dists, attn_dists, target_batch, dec_padding_mask):
    tgt_flat = target_batch.reshape(-1)
    out = _fused(tgt_flat,
                 final_dists.reshape(BT, V),
                 target_batch.reshape(BT, 1),
                 dec_padding_mask.reshape(B, 1, T),
                 attn_dists)
    return out.reshape(())
